# baseline (speedup 1.0000x reference)
import functools

import jax
import jax.numpy as jnp
from jax import lax
from jax.experimental import pallas as pl
from jax.experimental.pallas import tpu as pltpu
from jax.experimental.pallas import tpu_sc as plsc

_VOCAB = 100000
_DIM = 128
_BATCH = 16384
_NEG = 20
_LANES = 16

_NW = 32
_CB = _BATCH // _NW
_CH = 16
_NCH = _CB // _CH
_VN = _CH * (1 + _NEG)
_NROWS = 24


def _sc_body(pos_u_hbm, pos_v_hbm, neg_hbm, u_w_hbm, v_w_hbm, out_hbm,
             idx_u, idx_v, idx_n, idx_vn, u_buf, vn_buf, score_v,
             sem0, sem1):
    nc = 2
    wid = lax.axis_index("s") * nc + lax.axis_index("c")
    base = wid * _CB

    st1 = pltpu.async_copy(pos_u_hbm.at[pl.ds(base, _CB)], idx_u, sem0)
    st2 = pltpu.async_copy(pos_v_hbm.at[pl.ds(base, _CB)], idx_v, sem0)
    st3 = pltpu.async_copy(
        neg_hbm.at[pl.ds(base * _NEG, _CB * _NEG)], idx_n, sem1)
    st1.wait()
    st2.wait()
    st3.wait()

    def merge_body(c, carry):
        o = c * _VN
        idx_vn[pl.ds(o, _LANES)] = idx_v[pl.ds(c * _CH, _LANES)]
        for t in range(_CH * _NEG // _LANES):
            idx_vn[pl.ds(o + _CH + t * _LANES, _LANES)] = (
                idx_n[pl.ds(c * _CH * _NEG + t * _LANES, _LANES)])
        return carry

    lax.fori_loop(0, _NCH, merge_body, 0)

    sems = (sem0, sem1)

    def dmas(c, slot):
        return [
            (u_w_hbm.at[idx_u.at[pl.ds(c * _CH, _CH)]],
             u_buf.at[slot], sems[slot]),
            (v_w_hbm.at[idx_vn.at[pl.ds(c * _VN, _VN)]],
             vn_buf.at[slot], sems[slot]),
        ]

    def fire(c, slot):
        for s, d, m in dmas(c, slot):
            pltpu.async_copy(s, d, m)

    def drain(c, slot):
        for s, d, m in dmas(c, slot):
            pltpu.make_async_copy(s, d, m).wait()

    lane = lax.iota(jnp.int32, _LANES)
    last_lane = lane == (_LANES - 1)
    nvec = _DIM // _LANES

    def compute(c, slot):
        ub = u_buf.at[slot]
        vb = vn_buf.at[slot]

        def b_body(b, carry):
            pos = jnp.full((_LANES,), c * _CH, jnp.int32) + b
            nrow = b * _NEG + _CH
            rows = [b] + [nrow + k for k in range(_NEG)]
            accs = [None] * len(rows)
            for j in range(nvec):
                sl = pl.ds(j * _LANES, _LANES)
                u_j = ub[b, sl]
                for r, row in enumerate(rows):
                    t = u_j * vb[row, sl]
                    accs[r] = t if j == 0 else accs[r] + t
            for r in range(len(rows)):
                plsc.store_scatter(score_v, [pos + r * _CB],
                                   plsc.cumsum(accs[r]), mask=last_lane)
            return carry

        lax.fori_loop(0, _CH, b_body, 0)

    fire(0, 0)

    def outer(i, carry):
        c0 = 2 * i
        fire(c0 + 1, 1)
        drain(c0, 0)
        compute(c0, 0)

        @pl.when(i < _NCH // 2 - 1)
        def _():
            fire(c0 + 2, 0)

        drain(c0 + 1, 1)
        compute(c0 + 1, 1)
        return carry

    lax.fori_loop(0, _NCH // 2, outer, 0)

    zero = jnp.zeros((_LANES,), jnp.float32)
    for r in range(_NEG + 1, _NROWS):
        def pad_body(i, carry, r=r):
            score_v[pl.ds(r * _CB + i * _LANES, _LANES)] = zero
            return carry
        lax.fori_loop(0, _CB // _LANES, pad_body, 0)

    pltpu.sync_copy(score_v, out_hbm.at[wid])


_sc_scores = functools.partial(
    pl.kernel,
    out_type=jax.ShapeDtypeStruct((_NW, _NROWS * _CB), jnp.float32),
    mesh=plsc.VectorSubcoreMesh(core_axis_name="c", subcore_axis_name="s"),
    scratch_types=[
        pltpu.VMEM((_CB,), jnp.int32),
        pltpu.VMEM((_CB,), jnp.int32),
        pltpu.VMEM((_CB * _NEG,), jnp.int32),
        pltpu.VMEM((_NCH * _VN,), jnp.int32),
        pltpu.VMEM((2, _CH, _DIM), jnp.float32),
        pltpu.VMEM((2, _VN, _DIM), jnp.float32),
        pltpu.VMEM((_NROWS * _CB,), jnp.float32),
        pltpu.SemaphoreType.DMA,
        pltpu.SemaphoreType.DMA,
    ],
    compiler_params=pltpu.CompilerParams(needs_layout_passes=False),
)(_sc_body)


def _tc_loss_body(s_ref, o_ref):
    x = s_ref[...]
    col = lax.broadcasted_iota(jnp.int32, x.shape, 1)
    row = col // _CB
    y = jax.nn.log_sigmoid(jnp.where(row == 0, x, -x))
    y = jnp.where(row < _NEG + 1, y, 0.0)
    w = jnp.where(row == 0, 1.0 / _BATCH,
                  jnp.where(row < _NEG + 1, 1.0 / (_BATCH * _NEG), 0.0))
    o_ref[0, 0] = -jnp.sum(y * w.astype(jnp.float32))


_tc_loss = pl.pallas_call(
    _tc_loss_body,
    out_shape=jax.ShapeDtypeStruct((1, 1), jnp.float32),
    out_specs=pl.BlockSpec(memory_space=pltpu.SMEM),
)


@jax.jit
def kernel(pos_u, pos_v, neg_v, u_weight, v_weight):
    pos_u = pos_u.astype(jnp.int32)
    pos_v = pos_v.astype(jnp.int32)
    neg_flat = neg_v.astype(jnp.int32).reshape(-1)
    scores = _sc_scores(pos_u, pos_v, neg_flat, u_weight, v_weight)
    return _tc_loss(scores)[0, 0]


# speedup vs baseline: 1.0384x; 1.0384x over previous
import functools

import jax
import jax.numpy as jnp
from jax import lax
from jax.experimental import pallas as pl
from jax.experimental.pallas import tpu as pltpu
from jax.experimental.pallas import tpu_sc as plsc

_VOCAB = 100000
_DIM = 128
_BATCH = 16384
_NEG = 20
_LANES = 16

_NW = 32
_CB = _BATCH // _NW
_CH = 16
_NCH = _CB // _CH
_VN = _CH * (1 + _NEG)
_NROWS = 24


def _sc_body(pos_u_hbm, vn_idx_hbm, u_w_hbm, v_w_hbm, out_hbm,
             idx_u, idx_vn, u_buf, vn_buf, score_v, sem0, sem1):
    nc = 2
    wid = lax.axis_index("s") * nc + lax.axis_index("c")
    base = wid * _CB

    ib = wid * (_NCH * _VN)
    st1 = pltpu.async_copy(pos_u_hbm.at[pl.ds(base, _CH)],
                           idx_u.at[pl.ds(0, _CH)], sem0)
    st2 = pltpu.async_copy(vn_idx_hbm.at[pl.ds(ib, _VN)],
                           idx_vn.at[pl.ds(0, _VN)], sem0)
    st3 = pltpu.async_copy(pos_u_hbm.at[pl.ds(base + _CH, _CB - _CH)],
                           idx_u.at[pl.ds(_CH, _CB - _CH)], sem1)
    st4 = pltpu.async_copy(
        vn_idx_hbm.at[pl.ds(ib + _VN, (_NCH - 1) * _VN)],
        idx_vn.at[pl.ds(_VN, (_NCH - 1) * _VN)], sem1)
    st1.wait()
    st2.wait()

    sems = (sem0, sem1)

    def dmas(c, slot):
        return [
            (u_w_hbm.at[idx_u.at[pl.ds(c * _CH, _CH)]],
             u_buf.at[slot], sems[slot]),
            (v_w_hbm.at[idx_vn.at[pl.ds(c * _VN, _VN)]],
             vn_buf.at[slot], sems[slot]),
        ]

    def fire(c, slot):
        for s, d, m in dmas(c, slot):
            pltpu.async_copy(s, d, m)

    def drain(c, slot):
        for s, d, m in dmas(c, slot):
            pltpu.make_async_copy(s, d, m).wait()

    lane = lax.iota(jnp.int32, _LANES)
    last_lane = lane == (_LANES - 1)
    nvec = _DIM // _LANES

    def compute(c, slot):
        ub = u_buf.at[slot]
        vb = vn_buf.at[slot]

        def b_body(b, carry):
            pos = jnp.full((_LANES,), c * _CH, jnp.int32) + b
            nrow = b * _NEG + _CH
            rows = [b] + [nrow + k for k in range(_NEG)]
            accs = [None] * len(rows)
            for j in range(nvec):
                sl = pl.ds(j * _LANES, _LANES)
                u_j = ub[b, sl]
                for r, row in enumerate(rows):
                    t = u_j * vb[row, sl]
                    accs[r] = t if j == 0 else accs[r] + t
            for r in range(len(rows)):
                plsc.store_scatter(score_v, [pos + r * _CB],
                                   plsc.cumsum(accs[r]), mask=last_lane)
            return carry

        lax.fori_loop(0, _CH, b_body, 0)

    fire(0, 0)
    st3.wait()
    st4.wait()

    def outer(i, carry):
        c0 = 2 * i
        fire(c0 + 1, 1)
        drain(c0, 0)
        compute(c0, 0)

        @pl.when(i < _NCH // 2 - 1)
        def _():
            fire(c0 + 2, 0)

        drain(c0 + 1, 1)
        compute(c0 + 1, 1)
        return carry

    lax.fori_loop(0, _NCH // 2, outer, 0)

    zero = jnp.zeros((_LANES,), jnp.float32)
    for r in range(_NEG + 1, _NROWS):
        def pad_body(i, carry, r=r):
            score_v[pl.ds(r * _CB + i * _LANES, _LANES)] = zero
            return carry
        lax.fori_loop(0, _CB // _LANES, pad_body, 0)

    pltpu.sync_copy(score_v, out_hbm.at[wid])


_sc_scores = functools.partial(
    pl.kernel,
    out_type=jax.ShapeDtypeStruct((_NW, _NROWS * _CB), jnp.float32),
    mesh=plsc.VectorSubcoreMesh(core_axis_name="c", subcore_axis_name="s"),
    scratch_types=[
        pltpu.VMEM((_CB,), jnp.int32),
        pltpu.VMEM((_NCH * _VN,), jnp.int32),
        pltpu.VMEM((2, _CH, _DIM), jnp.float32),
        pltpu.VMEM((2, _VN, _DIM), jnp.float32),
        pltpu.VMEM((_NROWS * _CB,), jnp.float32),
        pltpu.SemaphoreType.DMA,
        pltpu.SemaphoreType.DMA,
    ],
    compiler_params=pltpu.CompilerParams(needs_layout_passes=False),
)(_sc_body)


def _tc_loss_body(s_ref, o_ref):
    x = s_ref[...]
    col = lax.broadcasted_iota(jnp.int32, x.shape, 1)
    row = col // _CB
    y = jax.nn.log_sigmoid(jnp.where(row == 0, x, -x))
    y = jnp.where(row < _NEG + 1, y, 0.0)
    w = jnp.where(row == 0, 1.0 / _BATCH,
                  jnp.where(row < _NEG + 1, 1.0 / (_BATCH * _NEG), 0.0))
    o_ref[0, 0] = -jnp.sum(y * w.astype(jnp.float32))


_tc_loss = pl.pallas_call(
    _tc_loss_body,
    out_shape=jax.ShapeDtypeStruct((1, 1), jnp.float32),
    out_specs=pl.BlockSpec(memory_space=pltpu.SMEM),
)


@jax.jit
def kernel(pos_u, pos_v, neg_v, u_weight, v_weight):
    pos_u = pos_u.astype(jnp.int32)
    pos_v = pos_v.astype(jnp.int32)
    neg_flat = neg_v.astype(jnp.int32).reshape(_BATCH // _CH, _CH * _NEG)
    vn_idx = jnp.concatenate(
        [pos_v.reshape(_BATCH // _CH, _CH), neg_flat], axis=1).reshape(-1)
    scores = _sc_scores(pos_u, vn_idx, u_weight, v_weight)
    return _tc_loss(scores)[0, 0]
